# baseline (speedup 1.0000x reference)
import jax
import jax.numpy as jnp
from jax import lax
from jax.experimental import pallas as pl
from jax.experimental.pallas import tpu as pltpu
from jax.experimental.pallas import tpu_sc as plsc

N_ROWS = 16384
N_CLS = 1000
RATE_KEEP = 13107

SC_ROWS = 6144
TC_ROWS = N_ROWS - SC_ROWS

BLOCK_ROWS = 2048
N_BLOCKS = TC_ROWS // BLOCK_ROWS
SUB_ROWS = BLOCK_ROWS // 128

_SC_NC = 2
_SC_NS = 16
_SC_NW = _SC_NC * _SC_NS
ROWS_PER_TILE = SC_ROWS // _SC_NW
SC_CH = 32
N_CHUNKS = ROWS_PER_TILE // SC_CH
GROUPS_PER_CH = SC_CH // 16


def _tc_ce_block(x_ref, t_ref, o_ref):
    x = x_ref[:]
    t = t_ref[:]
    m = jnp.max(x, axis=1, keepdims=True)
    s = jnp.sum(jnp.exp(x - m), axis=1, keepdims=True)
    lse = m + jnp.log(s)
    cls_ids = lax.broadcasted_iota(jnp.int32, x.shape, 1)
    tv = jnp.sum(jnp.where(cls_ids == t, x, 0.0), axis=1, keepdims=True)
    o_ref[:] = jnp.reshape(lse - tv, (SUB_ROWS, 128))


_N_FULL = 984 // 16
_TAIL0 = 976


def _sc_row(buf, row, t_r):
    lane = lax.iota(jnp.int32, 16)

    def max_body(j, m):
        return jnp.maximum(m, buf[row, pl.ds(j * 16, 16)])

    m16 = lax.fori_loop(0, _N_FULL, max_body,
                        buf[row, pl.ds(_TAIL0, 16)], unroll=4)
    tail2 = buf[row, pl.ds(N_CLS - 16, 16)]
    m16 = jnp.maximum(m16, tail2)
    m_r = plsc.cummax(m16)[15]

    def sum_body(j, s):
        return s + jnp.exp(buf[row, pl.ds(j * 16, 16)] - m_r)

    s16 = lax.fori_loop(0, _N_FULL, sum_body,
                        jnp.exp(buf[row, pl.ds(_TAIL0, 16)] - m_r),
                        unroll=4)
    s16 = s16 + jnp.where(lane >= 8, jnp.exp(tail2 - m_r), 0.0)
    s_r = plsc.cumsum(s16)[15]
    start = jnp.minimum((t_r // 16) * 16, N_CLS - 16)
    w = buf[row, pl.ds(start, 16)]
    tv_r = plsc.cummax(jnp.where(lane == t_r - start, w, -jnp.inf))[15]
    return m_r - tv_r, s_r


def _sc_ce(x_hbm, t_hbm, a_hbm, s_hbm, t_v, buf0, buf1, a_v, s_v,
           sem0, sem1, semt):
    wid = lax.axis_index("s") * _SC_NC + lax.axis_index("c")
    row0 = TC_ROWS + wid * ROWS_PER_TILE
    pltpu.async_copy(t_hbm.at[pl.ds(row0, ROWS_PER_TILE)], t_v, semt).wait()
    lane = lax.iota(jnp.int32, 16)

    bufs = (buf0, buf1)
    sems = (sem0, sem1)
    pending = [None, None]
    pending[0] = pltpu.async_copy(
        x_hbm.at[pl.ds(row0, SC_CH), :], bufs[0], sems[0])
    for c in range(N_CHUNKS):
        if c + 1 < N_CHUNKS:
            nb = (c + 1) % 2
            pending[nb] = pltpu.async_copy(
                x_hbm.at[pl.ds(row0 + (c + 1) * SC_CH, SC_CH), :],
                bufs[nb], sems[nb])
        pending[c % 2].wait()
        buf = bufs[c % 2]

        def group_body(g, _, _buf=buf, _c=c):
            off = _c * SC_CH + g * 16
            t16 = t_v[pl.ds(off, 16)]
            a16 = jnp.zeros((16,), jnp.float32)
            s16 = jnp.zeros((16,), jnp.float32)
            for rr in range(16):
                row = g * 16 + rr
                a_r, s_r = _sc_row(_buf, row, t16[rr])
                a16 = jnp.where(lane == rr, a_r, a16)
                s16 = jnp.where(lane == rr, s_r, s16)
            a_v[pl.ds(off, 16)] = a16
            s_v[pl.ds(off, 16)] = s16
            return 0

        lax.fori_loop(0, GROUPS_PER_CH, group_body, 0)

    out0 = wid * ROWS_PER_TILE
    pltpu.sync_copy(a_v, a_hbm.at[pl.ds(out0, ROWS_PER_TILE)])
    pltpu.sync_copy(s_v, s_hbm.at[pl.ds(out0, ROWS_PER_TILE)])


def _select_topk(p_ref, a_ref, s_ref, o_ref):
    k = RATE_KEEP
    v_tc = p_ref[:]
    v_sc = a_ref[:] + jnp.log(s_ref[:])
    v = jnp.concatenate([v_tc, v_sc], axis=0)
    bits = lax.bitcast_convert_type(v, jnp.int32)

    def body(_, carry):
        lo, hi = carry
        mid = lo + (hi - lo) // 2
        cnt = jnp.sum((bits >= mid).astype(jnp.int32))
        take = cnt >= k
        return jnp.where(take, mid, lo), jnp.where(take, hi, mid)

    lo, _ = lax.fori_loop(0, 31, body, (jnp.int32(0), jnp.int32(0x7F800000)))
    thr = lax.bitcast_convert_type(lo, jnp.float32)
    gt = bits > lo
    cnt_gt = jnp.sum(gt.astype(jnp.int32))
    sum_gt = jnp.sum(jnp.where(gt, v, 0.0))
    loss = (sum_gt + (k - cnt_gt).astype(jnp.float32) * thr) / k
    o_ref[:] = jnp.reshape(loss, (1, 1))


@jax.jit
def kernel(cls_pred, cls_target):
    tgt = cls_target.astype(jnp.int32)
    sc_ce = pl.kernel(
        _sc_ce,
        out_type=[
            jax.ShapeDtypeStruct((SC_ROWS,), jnp.float32),
            jax.ShapeDtypeStruct((SC_ROWS,), jnp.float32),
        ],
        mesh=plsc.VectorSubcoreMesh(core_axis_name="c", subcore_axis_name="s"),
        compiler_params=pltpu.CompilerParams(needs_layout_passes=False),
        scratch_types=[
            pltpu.VMEM((ROWS_PER_TILE,), jnp.int32),
            pltpu.VMEM((SC_CH, N_CLS), jnp.float32),
            pltpu.VMEM((SC_CH, N_CLS), jnp.float32),
            pltpu.VMEM((ROWS_PER_TILE,), jnp.float32),
            pltpu.VMEM((ROWS_PER_TILE,), jnp.float32),
            pltpu.SemaphoreType.DMA,
            pltpu.SemaphoreType.DMA,
            pltpu.SemaphoreType.DMA,
        ],
    )
    a_sc, s_sc = sc_ce(cls_pred, tgt)

    per_tc = pl.pallas_call(
        _tc_ce_block,
        grid=(N_BLOCKS,),
        in_specs=[
            pl.BlockSpec((BLOCK_ROWS, N_CLS), lambda i: (i, 0)),
            pl.BlockSpec((BLOCK_ROWS, 1), lambda i: (i, 0)),
        ],
        out_specs=pl.BlockSpec((SUB_ROWS, 128), lambda i: (i, 0)),
        out_shape=jax.ShapeDtypeStruct((TC_ROWS // 128, 128), jnp.float32),
    )(cls_pred, tgt.reshape(N_ROWS, 1))

    loss = pl.pallas_call(
        _select_topk,
        out_shape=jax.ShapeDtypeStruct((1, 1), jnp.float32),
    )(per_tc, a_sc.reshape(SC_ROWS // 128, 128), s_sc.reshape(SC_ROWS // 128, 128))
    return loss[0, 0]


# speedup vs baseline: 1.0011x; 1.0011x over previous
import jax
import jax.numpy as jnp
from jax import lax
from jax.experimental import pallas as pl
from jax.experimental.pallas import tpu as pltpu

N_ROWS = 16384
N_CLS = 1000
RATE_KEEP = 13107
BLOCK_ROWS = 2048
N_BLOCKS = N_ROWS // BLOCK_ROWS
SUB_ROWS = BLOCK_ROWS // 128


def _ohem_block(x_ref, t_ref, o_ref, acc_ref):
    t = t_ref[:]
    m = jnp.full((BLOCK_ROWS, 1), -jnp.inf, jnp.float32)
    s = jnp.zeros((BLOCK_ROWS, 1), jnp.float32)
    tv = jnp.zeros((BLOCK_ROWS, 1), jnp.float32)
    for c0 in range(0, N_CLS, 128):
        w = min(128, N_CLS - c0)
        xc = x_ref[:, pl.ds(c0, w)]
        mc = jnp.max(xc, axis=1, keepdims=True)
        m_new = jnp.maximum(m, mc)
        s = s * jnp.exp(m - m_new) + jnp.sum(
            jnp.exp(xc - m_new), axis=1, keepdims=True)
        cls_ids = lax.broadcasted_iota(jnp.int32, xc.shape, 1) + c0
        tv = tv + jnp.sum(jnp.where(cls_ids == t, xc, 0.0),
                          axis=1, keepdims=True)
        m = m_new
    per = m + jnp.log(s) - tv

    i = pl.program_id(0)
    acc_ref[pl.ds(i * SUB_ROWS, SUB_ROWS), :] = jnp.reshape(per, (SUB_ROWS, 128))

    @pl.when(i == N_BLOCKS - 1)
    def _select():
        k = RATE_KEEP
        v = acc_ref[:]
        bits = lax.bitcast_convert_type(v, jnp.int32)

        def body(_, carry):
            lo, hi = carry
            mid = lo + (hi - lo) // 2
            cnt = jnp.sum((bits >= mid).astype(jnp.int32))
            take = cnt >= k
            return jnp.where(take, mid, lo), jnp.where(take, hi, mid)

        lo, _ = lax.fori_loop(0, 31, body, (jnp.int32(0), jnp.int32(0x7F800000)))
        thr = lax.bitcast_convert_type(lo, jnp.float32)
        gt = bits > lo
        cnt_gt = jnp.sum(gt.astype(jnp.int32))
        sum_gt = jnp.sum(jnp.where(gt, v, 0.0))
        loss = (sum_gt + (k - cnt_gt).astype(jnp.float32) * thr) / k
        o_ref[:] = jnp.reshape(loss, (1, 1))


@jax.jit
def kernel(cls_pred, cls_target):
    tgt = cls_target.astype(jnp.int32).reshape(N_ROWS, 1)
    loss = pl.pallas_call(
        _ohem_block,
        grid=(N_BLOCKS,),
        in_specs=[
            pl.BlockSpec((BLOCK_ROWS, N_CLS), lambda i: (i, 0)),
            pl.BlockSpec((BLOCK_ROWS, 1), lambda i: (i, 0)),
        ],
        out_specs=pl.BlockSpec((1, 1), lambda i: (0, 0)),
        out_shape=jax.ShapeDtypeStruct((1, 1), jnp.float32),
        scratch_shapes=[pltpu.VMEM((128, 128), jnp.float32)],
    )(cls_pred, tgt)
    return loss[0, 0]


# speedup vs baseline: 1.3613x; 1.3598x over previous
import jax
import jax.numpy as jnp
from jax import lax
from jax.experimental import pallas as pl
from jax.experimental.pallas import tpu as pltpu

N_ROWS = 16384
N_CLS = 1000
RATE_KEEP = 13107
BLOCK_ROWS = 2048
N_BLOCKS = N_ROWS // BLOCK_ROWS
SUB_ROWS = BLOCK_ROWS // 128


def _ohem_block(x_ref, t_ref, o_ref, acc_ref):
    x = x_ref[:]
    t = t_ref[:]
    m = jnp.max(x, axis=1, keepdims=True)
    s = jnp.sum(jnp.exp(x - m), axis=1, keepdims=True)
    lse = m + jnp.log(s)
    cls_ids = lax.broadcasted_iota(jnp.int32, x.shape, 1)
    tv = jnp.sum(jnp.where(cls_ids == t, x, 0.0), axis=1, keepdims=True)
    per = lse - tv

    i = pl.program_id(0)
    acc_ref[pl.ds(i * SUB_ROWS, SUB_ROWS), :] = jnp.reshape(per, (SUB_ROWS, 128))

    @pl.when(i == N_BLOCKS - 1)
    def _select():
        k = RATE_KEEP
        v = acc_ref[:]
        bits = lax.bitcast_convert_type(v, jnp.int32)

        def body(_, carry):
            lo, hi = carry
            mid = lo + (hi - lo) // 2
            cnt = jnp.sum((bits >= mid).astype(jnp.int32))
            take = cnt >= k
            return jnp.where(take, mid, lo), jnp.where(take, hi, mid)

        lo, _ = lax.fori_loop(0, 31, body, (jnp.int32(0), jnp.int32(0x7F800000)))
        thr = lax.bitcast_convert_type(lo, jnp.float32)
        gt = bits > lo
        cnt_gt = jnp.sum(gt.astype(jnp.int32))
        sum_gt = jnp.sum(jnp.where(gt, v, 0.0))
        loss = (sum_gt + (k - cnt_gt).astype(jnp.float32) * thr) / k
        o_ref[:] = jnp.reshape(loss, (1, 1))


@jax.jit
def kernel(cls_pred, cls_target):
    tgt = cls_target.astype(jnp.int32).reshape(N_ROWS, 1)
    loss = pl.pallas_call(
        _ohem_block,
        grid=(N_BLOCKS,),
        in_specs=[
            pl.BlockSpec((BLOCK_ROWS, N_CLS), lambda i: (i, 0)),
            pl.BlockSpec((BLOCK_ROWS, 1), lambda i: (i, 0)),
        ],
        out_specs=pl.BlockSpec((1, 1), lambda i: (0, 0)),
        out_shape=jax.ShapeDtypeStruct((1, 1), jnp.float32),
        scratch_shapes=[pltpu.VMEM((128, 128), jnp.float32)],
    )(cls_pred, tgt)
    return loss[0, 0]
